# Initial kernel scaffold; baseline (speedup 1.0000x reference)
import jax
import jax.numpy as jnp
from jax.experimental import pallas as pl


def kernel(x, edge_index, edge_attr, We_d, Wn_d, Wg_d, We_p, Wn_p, Wg_p, We_s, Wn_s, Wg_s):
    raise NotImplementedError("write your pallas kernel here")


# speedup vs baseline: 4.8900x; 4.8900x over previous
import functools

import jax
import jax.numpy as jnp
from jax import lax
from jax.experimental import pallas as pl
from jax.experimental.pallas import tpu as pltpu
from jax.experimental.pallas import tpu_sc as plsc

N = 10000
E = 320000
D = 128
DE = 16
NC = 2
NS = 16
NW = NC * NS
GSUB = 128
CHUNK = 512
NSUB = CHUNK // GSUB
NCHUNKS = E // CHUNK
ROWS_PER_SUB = 632
NPAD = ROWS_PER_SUB * NS


def _relu(v):
    return jnp.maximum(v, 0.0)


def _make_edge_stage(nb):
    W = 16 * nb
    mesh = plsc.VectorSubcoreMesh(
        core_axis_name="c", subcore_axis_name="s", num_cores=NC, num_subcores=NS
    )

    @functools.partial(
        pl.kernel,
        mesh=mesh,
        compiler_params=pltpu.CompilerParams(use_tc_tiling_on_sc=False),
        out_type=(
            jax.ShapeDtypeStruct((E, 16), jnp.float32),
            jax.ShapeDtypeStruct((NC, NPAD, W), jnp.float32),
        ),
        scratch_types=[
            pltpu.VMEM((2 * NSUB, GSUB), jnp.int32),
            pltpu.VMEM((CHUNK, W), jnp.float32),
            pltpu.VMEM((CHUNK, W), jnp.float32),
            pltpu.VMEM((CHUNK, W), jnp.float32),
            pltpu.VMEM((CHUNK, W), jnp.float32),
            pltpu.VMEM((CHUNK, 16), jnp.float32),
            pltpu.VMEM_SHARED((NPAD, W), jnp.float32),
            pltpu.SemaphoreType.DMA,
        ],
    )
    def stage(idx_r, stab, dtab, ea, zrows,
              ef_out, agg_out,
              sdidx, sv, dv, eav, efv, efo, agg_sh, sem):
        cid = lax.axis_index("c")
        sid = lax.axis_index("s")
        wid = sid * NC + cid
        r0 = sid * ROWS_PER_SUB
        pltpu.sync_copy(zrows.at[pl.ds(r0, ROWS_PER_SUB)],
                        agg_sh.at[pl.ds(r0, ROWS_PER_SUB)])
        plsc.subcore_barrier()

        nk = 19 + jnp.where(wid < NCHUNKS - 19 * NW, 1, 0)

        def chunk_body(k, carry):
            q = wid + k * NW
            eb = q * CHUNK
            pltpu.sync_copy(idx_r.at[q], sdidx)
            cps = []
            for j in range(NSUB):
                cps.append(pltpu.async_copy(
                    stab.at[sdidx.at[j]], sv.at[pl.ds(j * GSUB, GSUB)], sem))
                cps.append(pltpu.async_copy(
                    dtab.at[sdidx.at[NSUB + j]], dv.at[pl.ds(j * GSUB, GSUB)], sem))
            pltpu.sync_copy(ea.at[pl.ds(eb, CHUNK)], eav)
            for cp in cps:
                cp.wait()

            if nb == 2:
                def edge_body(e, c2):
                    e1 = _relu(sv[e, pl.ds(0, 16)] + dv[e, pl.ds(0, 16)]
                               + eav[e, pl.ds(0, 16)])
                    e2 = _relu(sv[e, pl.ds(16, 16)] + dv[e, pl.ds(16, 16)]
                               + eav[e, pl.ds(16, 16)])
                    efv[e, pl.ds(0, 16)] = e1
                    efv[e, pl.ds(16, 16)] = e2
                    efo[e, :] = e1 + e2
                    return c2
            else:
                def edge_body(e, c2):
                    efv[e, :] = _relu(sv[e, :] + dv[e, :] + eav[e, :])
                    return c2
            lax.fori_loop(0, CHUNK, edge_body, 0, unroll=8)

            for j in range(NSUB):
                pltpu.sync_copy(efv.at[pl.ds(j * GSUB, GSUB)],
                                agg_sh.at[sdidx.at[NSUB + j]], add=True)
            out_src = efo if nb == 2 else efv
            pltpu.sync_copy(out_src, ef_out.at[pl.ds(eb, CHUNK)])
            return carry

        lax.fori_loop(0, nk, chunk_body, 0)
        plsc.subcore_barrier()
        pltpu.sync_copy(agg_sh.at[pl.ds(r0, ROWS_PER_SUB)],
                        agg_out.at[cid, pl.ds(r0, ROWS_PER_SUB)])

    return stage


_edge_stage2 = _make_edge_stage(2)
_edge_stage1 = _make_edge_stage(1)


def _tc_node_pre(x, wa, wb, wc):
    def body(x_ref, wa_ref, wb_ref, wc_ref, o1, o2, o3):
        xv = x_ref[...]
        o1[...] = jnp.dot(xv, wa_ref[...], preferred_element_type=jnp.float32)
        o2[...] = jnp.dot(xv, wb_ref[...], preferred_element_type=jnp.float32)
        o3[...] = jnp.dot(xv, wc_ref[...], preferred_element_type=jnp.float32)
    return pl.pallas_call(
        body,
        out_shape=(
            jax.ShapeDtypeStruct((N, 256), jnp.float32),
            jax.ShapeDtypeStruct((N, 32), jnp.float32),
            jax.ShapeDtypeStruct((N, 32), jnp.float32),
        ),
    )(x, wa, wb, wc)


def _tc_edge_mm(a, w):
    K = a.shape[1]
    M = w.shape[1]
    BR = 8000
    def body(a_ref, w_ref, o_ref):
        o_ref[...] = jnp.dot(a_ref[...], w_ref[...],
                             preferred_element_type=jnp.float32)
    return pl.pallas_call(
        body,
        grid=(E // BR,),
        in_specs=[pl.BlockSpec((BR, K), lambda i: (i, 0)),
                  pl.BlockSpec((K, M), lambda i: (0, 0))],
        out_specs=pl.BlockSpec((BR, M), lambda i: (i, 0)),
        out_shape=jax.ShapeDtypeStruct((E, M), jnp.float32),
    )(a, w)


def _tc_mid(aggdp, xwn, wnd2, wnp2, wgd, wgp, wss, wns1):
    def body(agg_ref, xwn_ref, wnd2_ref, wnp2_ref, wgd_ref, wgp_ref,
             wss_ref, wns1_ref, ns_o, nd_o, xwns_o, gfp_o):
        a = agg_ref[...]
        agg = a[0, :N] + a[1, :N]
        xw = xwn_ref[...]
        nf_d = _relu(xw[:, 0:128] + jnp.dot(
            agg[:, 0:16], wnd2_ref[...], preferred_element_type=jnp.float32))
        nf_p = _relu(xw[:, 128:256] + jnp.dot(
            agg[:, 16:32], wnp2_ref[...], preferred_element_type=jnp.float32))
        nf = nf_d + nf_p
        nsnd = jnp.dot(nf, wss_ref[...], preferred_element_type=jnp.float32)
        ns_o[...] = nsnd[:, 0:16]
        nd_o[...] = nsnd[:, 16:32]
        xwns_o[...] = jnp.dot(nf, wns1_ref[...],
                              preferred_element_type=jnp.float32)
        md = jnp.sum(nf_d, axis=0, keepdims=True) * (1.0 / N)
        mp = jnp.sum(nf_p, axis=0, keepdims=True) * (1.0 / N)
        gfp_o[...] = (jnp.dot(md, wgd_ref[...], preferred_element_type=jnp.float32)
                      + jnp.dot(mp, wgp_ref[...], preferred_element_type=jnp.float32))
    return pl.pallas_call(
        body,
        out_shape=(
            jax.ShapeDtypeStruct((N, 16), jnp.float32),
            jax.ShapeDtypeStruct((N, 16), jnp.float32),
            jax.ShapeDtypeStruct((N, 128), jnp.float32),
            jax.ShapeDtypeStruct((1, 128), jnp.float32),
        ),
    )(aggdp, xwn, wnd2, wnp2, wgd, wgp, wss, wns1)


def _tc_post(aggs, xwns, wns2, wgs, gfp):
    def body(aggs_ref, xwns_ref, wns2_ref, wgs_ref, gfp_ref, nf_o, gf_o):
        a = aggs_ref[...]
        agg = a[0, :N] + a[1, :N]
        nf_s = _relu(xwns_ref[...] + jnp.dot(
            agg, wns2_ref[...], preferred_element_type=jnp.float32))
        nf_o[...] = nf_s
        ms = jnp.sum(nf_s, axis=0, keepdims=True) * (1.0 / N)
        gf_o[...] = gfp_ref[...] + jnp.dot(
            ms, wgs_ref[...], preferred_element_type=jnp.float32)
    return pl.pallas_call(
        body,
        out_shape=(
            jax.ShapeDtypeStruct((N, 128), jnp.float32),
            jax.ShapeDtypeStruct((1, 128), jnp.float32),
        ),
    )(aggs, xwns, wns2, wgs, gfp)


def kernel(x, edge_index, edge_attr,
           We_d, Wn_d, Wg_d, We_p, Wn_p, Wg_p, We_s, Wn_s, Wg_s):
    ei = edge_index.astype(jnp.int32)
    idx_r = jnp.concatenate(
        [ei[0].reshape(NCHUNKS, NSUB, GSUB),
         ei[1].reshape(NCHUNKS, NSUB, GSUB)], axis=1)

    wa = jnp.concatenate([Wn_d[:128], Wn_p[:128]], axis=1)
    wb = jnp.concatenate([We_d[:128], We_p[:128]], axis=1)
    wc = jnp.concatenate([We_d[128:256], We_p[128:256]], axis=1)
    wedge = jnp.concatenate([We_d[256:], We_p[256:]], axis=1)

    xwn, xs_tab, xd_tab = _tc_node_pre(x, wa, wb, wc)
    ea_dp = _tc_edge_mm(edge_attr, wedge)

    z32 = jnp.zeros((NPAD, 32), jnp.float32)
    ef_in, agg_dp = _edge_stage2(idx_r, xs_tab, xd_tab, ea_dp, z32)

    wss = jnp.concatenate([We_s[:128], We_s[128:256]], axis=1)
    ns_tab, nd_tab, xwns, gfp = _tc_mid(
        agg_dp, xwn, Wn_d[128:], Wn_p[128:], Wg_d, Wg_p, wss, Wn_s[:128])
    es_in = _tc_edge_mm(ef_in, We_s[256:])

    z16 = jnp.zeros((NPAD, 16), jnp.float32)
    ef_s, agg_s = _edge_stage1(idx_r, ns_tab, nd_tab, es_in, z16)

    nf_s, gf = _tc_post(agg_s, xwns, Wn_s[128:], Wg_s, gfp)
    return nf_s, ef_s, gf.reshape(D)


# speedup vs baseline: 7.9070x; 1.6170x over previous
import functools

import jax
import jax.numpy as jnp
from jax import lax
from jax.experimental import pallas as pl
from jax.experimental.pallas import tpu as pltpu
from jax.experimental.pallas import tpu_sc as plsc

N = 10000
E = 320000
D = 128
DE = 16
NC = 2
NS = 16
NW = NC * NS
GSUB = 128
CHUNK = 512
NSUB = CHUNK // GSUB
RPC = CHUNK // 8
NCHUNKS = E // CHUNK
ROWS_PER_SUB = 632
NPAD = ROWS_PER_SUB * NS
ER = E // 8


def _relu(v):
    return jnp.maximum(v, 0.0)


def _make_edge_stage(nb):
    W = 16 * nb
    mesh = plsc.VectorSubcoreMesh(
        core_axis_name="c", subcore_axis_name="s", num_cores=NC, num_subcores=NS
    )

    ea_scratch = [pltpu.VMEM((RPC, 128), jnp.float32) for _ in range(nb)]

    @functools.partial(
        pl.kernel,
        mesh=mesh,
        compiler_params=pltpu.CompilerParams(use_tc_tiling_on_sc=False),
        out_type=(
            jax.ShapeDtypeStruct((ER, 128), jnp.float32),
            jax.ShapeDtypeStruct((NC, NPAD, W), jnp.float32),
        ),
        scratch_types=[
            pltpu.VMEM((2 * NSUB, GSUB), jnp.int32),
            pltpu.VMEM((CHUNK, W), jnp.float32),
            pltpu.VMEM((CHUNK, W), jnp.float32),
            *ea_scratch,
            pltpu.VMEM((CHUNK, W), jnp.float32),
            pltpu.VMEM((RPC, 128), jnp.float32),
            pltpu.VMEM_SHARED((NPAD, W), jnp.float32),
            pltpu.SemaphoreType.DMA,
        ],
    )
    def stage(idx_r, stab, dtab, *rest):
        if nb == 2:
            (ea0, ea1, zrows, ef_out, agg_out,
             sdidx, sv, dv, eav0, eav1, efv, efo, agg_sh, sem) = rest
            eas, eavs = (ea0, ea1), (eav0, eav1)
        else:
            (ea0, zrows, ef_out, agg_out,
             sdidx, sv, dv, eav0, efv, efo, agg_sh, sem) = rest
            eas, eavs = (ea0,), (eav0,)
        cid = lax.axis_index("c")
        sid = lax.axis_index("s")
        wid = sid * NC + cid
        r0 = sid * ROWS_PER_SUB
        pltpu.sync_copy(zrows.at[pl.ds(r0, ROWS_PER_SUB)],
                        agg_sh.at[pl.ds(r0, ROWS_PER_SUB)])
        plsc.subcore_barrier()

        nk = 19 + jnp.where(wid < NCHUNKS - 19 * NW, 1, 0)

        def chunk_body(k, carry):
            q = wid + k * NW
            rb = q * RPC
            pltpu.sync_copy(idx_r.at[q], sdidx)
            cps = []
            for j in range(NSUB):
                cps.append(pltpu.async_copy(
                    stab.at[sdidx.at[j]], sv.at[pl.ds(j * GSUB, GSUB)], sem))
                cps.append(pltpu.async_copy(
                    dtab.at[sdidx.at[NSUB + j]], dv.at[pl.ds(j * GSUB, GSUB)],
                    sem))
            for b in range(nb):
                pltpu.sync_copy(eas[b].at[pl.ds(rb, RPC)], eavs[b])
            for cp in cps:
                cp.wait()

            def row_body(rr, c2):
                for jj in range(8):
                    e = rr * 8 + jj
                    acc = None
                    for b in range(nb):
                        s = sv[e, pl.ds(16 * b, 16)]
                        d = dv[e, pl.ds(16 * b, 16)]
                        a = eavs[b][rr, pl.ds(16 * jj, 16)]
                        efb = _relu(s + d + a)
                        efv[e, pl.ds(16 * b, 16)] = efb
                        acc = efb if acc is None else acc + efb
                    efo[rr, pl.ds(16 * jj, 16)] = acc
                return c2
            lax.fori_loop(0, RPC, row_body, 0)

            for j in range(NSUB):
                pltpu.sync_copy(efv.at[pl.ds(j * GSUB, GSUB)],
                                agg_sh.at[sdidx.at[NSUB + j]], add=True)
            pltpu.sync_copy(efo, ef_out.at[pl.ds(rb, RPC)])
            return carry

        lax.fori_loop(0, nk, chunk_body, 0)
        plsc.subcore_barrier()
        pltpu.sync_copy(agg_sh.at[pl.ds(r0, ROWS_PER_SUB)],
                        agg_out.at[cid, pl.ds(r0, ROWS_PER_SUB)])

    return stage


_edge_stage2 = _make_edge_stage(2)
_edge_stage1 = _make_edge_stage(1)


def _tc_node_pre(x, wa, wb, wc):
    def body(x_ref, wa_ref, wb_ref, wc_ref, o1, o2, o3):
        xv = x_ref[...]
        o1[...] = jnp.dot(xv, wa_ref[...], preferred_element_type=jnp.float32)
        o2[...] = jnp.dot(xv, wb_ref[...], preferred_element_type=jnp.float32)
        o3[...] = jnp.dot(xv, wc_ref[...], preferred_element_type=jnp.float32)
    return pl.pallas_call(
        body,
        out_shape=(
            jax.ShapeDtypeStruct((N, 256), jnp.float32),
            jax.ShapeDtypeStruct((N, 32), jnp.float32),
            jax.ShapeDtypeStruct((N, 32), jnp.float32),
        ),
    )(x, wa, wb, wc)


def _tc_edge_bd2(a, w0, w1):
    BR = 8000
    def body(a_ref, w0_ref, w1_ref, o0_ref, o1_ref):
        av = a_ref[...]
        o0_ref[...] = jnp.dot(av, w0_ref[...],
                              preferred_element_type=jnp.float32)
        o1_ref[...] = jnp.dot(av, w1_ref[...],
                              preferred_element_type=jnp.float32)
    return pl.pallas_call(
        body,
        grid=(ER // BR,),
        in_specs=[pl.BlockSpec((BR, 128), lambda i: (i, 0)),
                  pl.BlockSpec((128, 128), lambda i: (0, 0)),
                  pl.BlockSpec((128, 128), lambda i: (0, 0))],
        out_specs=(pl.BlockSpec((BR, 128), lambda i: (i, 0)),
                   pl.BlockSpec((BR, 128), lambda i: (i, 0))),
        out_shape=(jax.ShapeDtypeStruct((ER, 128), jnp.float32),
                   jax.ShapeDtypeStruct((ER, 128), jnp.float32)),
    )(a, w0, w1)


def _tc_edge_bd1(a, w0):
    BR = 8000
    def body(a_ref, w0_ref, o0_ref):
        o0_ref[...] = jnp.dot(a_ref[...], w0_ref[...],
                              preferred_element_type=jnp.float32)
    return pl.pallas_call(
        body,
        grid=(ER // BR,),
        in_specs=[pl.BlockSpec((BR, 128), lambda i: (i, 0)),
                  pl.BlockSpec((128, 128), lambda i: (0, 0))],
        out_specs=pl.BlockSpec((BR, 128), lambda i: (i, 0)),
        out_shape=jax.ShapeDtypeStruct((ER, 128), jnp.float32),
    )(a, w0)


def _tc_mid(aggdp, xwn, wnd2, wnp2, wgd, wgp, wss, wns1):
    def body(agg_ref, xwn_ref, wnd2_ref, wnp2_ref, wgd_ref, wgp_ref,
             wss_ref, wns1_ref, ns_o, nd_o, xwns_o, gfp_o):
        a = agg_ref[...]
        agg = a[0, :N] + a[1, :N]
        xw = xwn_ref[...]
        nf_d = _relu(xw[:, 0:128] + jnp.dot(
            agg[:, 0:16], wnd2_ref[...], preferred_element_type=jnp.float32))
        nf_p = _relu(xw[:, 128:256] + jnp.dot(
            agg[:, 16:32], wnp2_ref[...], preferred_element_type=jnp.float32))
        nf = nf_d + nf_p
        nsnd = jnp.dot(nf, wss_ref[...], preferred_element_type=jnp.float32)
        ns_o[...] = nsnd[:, 0:16]
        nd_o[...] = nsnd[:, 16:32]
        xwns_o[...] = jnp.dot(nf, wns1_ref[...],
                              preferred_element_type=jnp.float32)
        md = jnp.sum(nf_d, axis=0, keepdims=True) * (1.0 / N)
        mp = jnp.sum(nf_p, axis=0, keepdims=True) * (1.0 / N)
        gfp_o[...] = (jnp.dot(md, wgd_ref[...], preferred_element_type=jnp.float32)
                      + jnp.dot(mp, wgp_ref[...], preferred_element_type=jnp.float32))
    return pl.pallas_call(
        body,
        out_shape=(
            jax.ShapeDtypeStruct((N, 16), jnp.float32),
            jax.ShapeDtypeStruct((N, 16), jnp.float32),
            jax.ShapeDtypeStruct((N, 128), jnp.float32),
            jax.ShapeDtypeStruct((1, 128), jnp.float32),
        ),
    )(aggdp, xwn, wnd2, wnp2, wgd, wgp, wss, wns1)


def _tc_post(aggs, xwns, wns2, wgs, gfp):
    def body(aggs_ref, xwns_ref, wns2_ref, wgs_ref, gfp_ref, nf_o, gf_o):
        a = aggs_ref[...]
        agg = a[0, :N] + a[1, :N]
        nf_s = _relu(xwns_ref[...] + jnp.dot(
            agg, wns2_ref[...], preferred_element_type=jnp.float32))
        nf_o[...] = nf_s
        ms = jnp.sum(nf_s, axis=0, keepdims=True) * (1.0 / N)
        gf_o[...] = gfp_ref[...] + jnp.dot(
            ms, wgs_ref[...], preferred_element_type=jnp.float32)
    return pl.pallas_call(
        body,
        out_shape=(
            jax.ShapeDtypeStruct((N, 128), jnp.float32),
            jax.ShapeDtypeStruct((1, 128), jnp.float32),
        ),
    )(aggs, xwns, wns2, wgs, gfp)


def kernel(x, edge_index, edge_attr,
           We_d, Wn_d, Wg_d, We_p, Wn_p, Wg_p, We_s, Wn_s, Wg_s):
    ei = edge_index.astype(jnp.int32)
    idx_r = jnp.concatenate(
        [ei[0].reshape(NCHUNKS, NSUB, GSUB),
         ei[1].reshape(NCHUNKS, NSUB, GSUB)], axis=1)
    ea_r = edge_attr.reshape(ER, 128)

    wa = jnp.concatenate([Wn_d[:128], Wn_p[:128]], axis=1)
    wb = jnp.concatenate([We_d[:128], We_p[:128]], axis=1)
    wc = jnp.concatenate([We_d[128:256], We_p[128:256]], axis=1)
    eye8 = jnp.eye(8, dtype=jnp.float32)
    w3d = jnp.kron(eye8, We_d[256:])
    w3p = jnp.kron(eye8, We_p[256:])
    w3s = jnp.kron(eye8, We_s[256:])

    xwn, xs_tab, xd_tab = _tc_node_pre(x, wa, wb, wc)
    ea_d, ea_p = _tc_edge_bd2(ea_r, w3d, w3p)

    z32 = jnp.zeros((NPAD, 32), jnp.float32)
    ef_in, agg_dp = _edge_stage2(idx_r, xs_tab, xd_tab, ea_d, ea_p, z32)

    wss = jnp.concatenate([We_s[:128], We_s[128:256]], axis=1)
    ns_tab, nd_tab, xwns, gfp = _tc_mid(
        agg_dp, xwn, Wn_d[128:], Wn_p[128:], Wg_d, Wg_p, wss, Wn_s[:128])
    es_in = _tc_edge_bd1(ef_in, w3s)

    z16 = jnp.zeros((NPAD, 16), jnp.float32)
    ef_s, agg_s = _edge_stage1(idx_r, ns_tab, nd_tab, es_in, z16)

    nf_s, gf = _tc_post(agg_s, xwns, Wn_s[128:], Wg_s, gfp)
    return nf_s, ef_s.reshape(E, DE), gf.reshape(D)


# speedup vs baseline: 8.6757x; 1.0972x over previous
import functools

import jax
import jax.numpy as jnp
from jax import lax
from jax.experimental import pallas as pl
from jax.experimental.pallas import tpu as pltpu
from jax.experimental.pallas import tpu_sc as plsc

N = 10000
E = 320000
D = 128
DE = 16
NC = 2
NS = 16
NW = NC * NS
GSUB = 128
CHUNK = 256
NSUB = CHUNK // GSUB
RPC = CHUNK // 8
ROWS_PER_SUB = 632
NPAD = ROWS_PER_SUB * NS
ER = E // 8
KPW = 40
EPAD = KPW * NW * CHUNK
NCHP = EPAD // CHUNK
ERP = EPAD // 8


def _relu(v):
    return jnp.maximum(v, 0.0)


def _make_edge_stage(nb):
    W = 16 * nb
    mesh = plsc.VectorSubcoreMesh(
        core_axis_name="c", subcore_axis_name="s", num_cores=NC, num_subcores=NS
    )

    ea_scratch = [pltpu.VMEM((2, RPC, 128), jnp.float32) for _ in range(nb)]

    @functools.partial(
        pl.kernel,
        mesh=mesh,
        compiler_params=pltpu.CompilerParams(use_tc_tiling_on_sc=False),
        out_type=(
            jax.ShapeDtypeStruct((ERP, 128), jnp.float32),
            jax.ShapeDtypeStruct((NC, NPAD, W), jnp.float32),
        ),
        scratch_types=[
            pltpu.VMEM((KPW, 2 * NSUB, GSUB), jnp.int32),
            pltpu.VMEM((2, CHUNK, W), jnp.float32),
            pltpu.VMEM((2, CHUNK, W), jnp.float32),
            *ea_scratch,
            pltpu.VMEM((2, CHUNK, W), jnp.float32),
            pltpu.VMEM((2, RPC, 128), jnp.float32),
            pltpu.VMEM_SHARED((NPAD, W), jnp.float32),
            pltpu.SemaphoreType.DMA,
            pltpu.SemaphoreType.DMA,
            pltpu.SemaphoreType.DMA,
            pltpu.SemaphoreType.DMA,
        ],
    )
    def stage(idx_r, stab, dtab, *rest):
        if nb == 2:
            (ea0, ea1, zrows, ef_out, agg_out, idx_all, sv, dv, eav0, eav1,
             efv, efo, agg_sh, sg0, sg1, so0, so1) = rest
            eas, eavs = (ea0, ea1), (eav0, eav1)
        else:
            (ea0, zrows, ef_out, agg_out, idx_all, sv, dv, eav0,
             efv, efo, agg_sh, sg0, sg1, so0, so1) = rest
            eas, eavs = (ea0,), (eav0,)
        semg = (sg0, sg1)
        semo = (so0, so1)
        cid = lax.axis_index("c")
        sid = lax.axis_index("s")
        wid = sid * NC + cid
        q0 = wid * KPW
        r0 = sid * ROWS_PER_SUB
        pltpu.sync_copy(zrows.at[pl.ds(r0, ROWS_PER_SUB)],
                        agg_sh.at[pl.ds(r0, ROWS_PER_SUB)])
        pltpu.sync_copy(idx_r.at[pl.ds(q0, KPW)], idx_all)
        plsc.subcore_barrier()

        def fire_in(i, b):
            for j in range(NSUB):
                pltpu.async_copy(stab.at[idx_all.at[i, j]],
                                 sv.at[b, pl.ds(j * GSUB, GSUB)], semg[b])
                pltpu.async_copy(dtab.at[idx_all.at[i, NSUB + j]],
                                 dv.at[b, pl.ds(j * GSUB, GSUB)], semg[b])
            for t in range(nb):
                pltpu.async_copy(eas[t].at[pl.ds((q0 + i) * RPC, RPC)],
                                 eavs[t].at[b], semg[b])

        def drain_in(i, b):
            for j in range(NSUB):
                pltpu.make_async_copy(stab.at[idx_all.at[i, j]],
                                      sv.at[b, pl.ds(j * GSUB, GSUB)],
                                      semg[b]).wait()
                pltpu.make_async_copy(dtab.at[idx_all.at[i, NSUB + j]],
                                      dv.at[b, pl.ds(j * GSUB, GSUB)],
                                      semg[b]).wait()
            for t in range(nb):
                pltpu.make_async_copy(eas[t].at[pl.ds((q0 + i) * RPC, RPC)],
                                      eavs[t].at[b], semg[b]).wait()

        def fire_out(i, b):
            for j in range(NSUB):
                pltpu.sync_copy(efv.at[b, pl.ds(j * GSUB, GSUB)],
                                agg_sh.at[idx_all.at[i, NSUB + j]], add=True)
            pltpu.async_copy(efo.at[b], ef_out.at[pl.ds((q0 + i) * RPC, RPC)],
                             semo[b])

        def drain_out(i, b):
            pltpu.make_async_copy(efo.at[b],
                                  ef_out.at[pl.ds((q0 + i) * RPC, RPC)],
                                  semo[b]).wait()

        def compute(i, b):
            def row_body(rr, c2):
                for jj in range(8):
                    e = rr * 8 + jj
                    acc = None
                    for t in range(nb):
                        s = sv[b, e, pl.ds(16 * t, 16)]
                        d = dv[b, e, pl.ds(16 * t, 16)]
                        a = eavs[t][b, rr, pl.ds(16 * jj, 16)]
                        eft = _relu(s + d + a)
                        efv[b, e, pl.ds(16 * t, 16)] = eft
                        acc = eft if acc is None else acc + eft
                    efo[b, rr, pl.ds(16 * jj, 16)] = acc
                return c2
            lax.fori_loop(0, RPC, row_body, 0)

        fire_in(0, 0)
        fire_in(1, 1)
        drain_in(0, 0)
        compute(0, 0)
        fire_out(0, 0)
        fire_in(2, 0)
        drain_in(1, 1)
        compute(1, 1)
        fire_out(1, 1)
        fire_in(3, 1)

        def pair_body(kk, carry):
            for b in range(2):
                i = 2 * kk + b
                drain_in(i, b)
                drain_out(i - 2, b)
                compute(i, b)
                fire_out(i, b)
                fire_in(i + 2, b)
            return carry
        lax.fori_loop(1, KPW // 2 - 1, pair_body, 0)

        for b in range(2):
            i = KPW - 2 + b
            drain_in(i, b)
            drain_out(i - 2, b)
            compute(i, b)
            fire_out(i, b)
        for b in range(2):
            drain_out(KPW - 2 + b, b)

        plsc.subcore_barrier()
        pltpu.sync_copy(agg_sh.at[pl.ds(r0, ROWS_PER_SUB)],
                        agg_out.at[cid, pl.ds(r0, ROWS_PER_SUB)])

    return stage


_edge_stage2 = _make_edge_stage(2)
_edge_stage1 = _make_edge_stage(1)


def _tc_node_pre(x, wa, wb, wc):
    def body(x_ref, wa_ref, wb_ref, wc_ref, o1, o2, o3):
        xv = x_ref[...]
        o1[...] = jnp.dot(xv, wa_ref[...], preferred_element_type=jnp.float32)
        o2[...] = jnp.dot(xv, wb_ref[...], preferred_element_type=jnp.float32)
        o3[...] = jnp.dot(xv, wc_ref[...], preferred_element_type=jnp.float32)
    return pl.pallas_call(
        body,
        out_shape=(
            jax.ShapeDtypeStruct((N, 256), jnp.float32),
            jax.ShapeDtypeStruct((N, 32), jnp.float32),
            jax.ShapeDtypeStruct((N, 32), jnp.float32),
        ),
    )(x, wa, wb, wc)


def _tc_edge_bd2(a, w0, w1):
    BR = 8000
    def body(a_ref, w0_ref, w1_ref, o0_ref, o1_ref):
        av = a_ref[...]
        o0_ref[...] = jnp.dot(av, w0_ref[...],
                              preferred_element_type=jnp.float32)
        o1_ref[...] = jnp.dot(av, w1_ref[...],
                              preferred_element_type=jnp.float32)
    return pl.pallas_call(
        body,
        grid=(ER // BR,),
        in_specs=[pl.BlockSpec((BR, 128), lambda i: (i, 0)),
                  pl.BlockSpec((128, 128), lambda i: (0, 0)),
                  pl.BlockSpec((128, 128), lambda i: (0, 0))],
        out_specs=(pl.BlockSpec((BR, 128), lambda i: (i, 0)),
                   pl.BlockSpec((BR, 128), lambda i: (i, 0))),
        out_shape=(jax.ShapeDtypeStruct((ERP, 128), jnp.float32),
                   jax.ShapeDtypeStruct((ERP, 128), jnp.float32)),
    )(a, w0, w1)


def _tc_edge_bd1(a, w0):
    BR = 8000
    def body(a_ref, w0_ref, o0_ref):
        o0_ref[...] = jnp.dot(a_ref[...], w0_ref[...],
                              preferred_element_type=jnp.float32)
    return pl.pallas_call(
        body,
        grid=(ER // BR,),
        in_specs=[pl.BlockSpec((BR, 128), lambda i: (i, 0)),
                  pl.BlockSpec((128, 128), lambda i: (0, 0))],
        out_specs=pl.BlockSpec((BR, 128), lambda i: (i, 0)),
        out_shape=jax.ShapeDtypeStruct((ERP, 128), jnp.float32),
    )(a, w0)


def _tc_mid(aggdp, xwn, wnd2, wnp2, wgd, wgp, wss, wns1):
    def body(agg_ref, xwn_ref, wnd2_ref, wnp2_ref, wgd_ref, wgp_ref,
             wss_ref, wns1_ref, ns_o, nd_o, xwns_o, gfp_o):
        a = agg_ref[...]
        agg = a[0, :N] + a[1, :N]
        xw = xwn_ref[...]
        nf_d = _relu(xw[:, 0:128] + jnp.dot(
            agg[:, 0:16], wnd2_ref[...], preferred_element_type=jnp.float32))
        nf_p = _relu(xw[:, 128:256] + jnp.dot(
            agg[:, 16:32], wnp2_ref[...], preferred_element_type=jnp.float32))
        nf = nf_d + nf_p
        nsnd = jnp.dot(nf, wss_ref[...], preferred_element_type=jnp.float32)
        ns_o[...] = nsnd[:, 0:16]
        nd_o[...] = nsnd[:, 16:32]
        xwns_o[...] = jnp.dot(nf, wns1_ref[...],
                              preferred_element_type=jnp.float32)
        md = jnp.sum(nf_d, axis=0, keepdims=True) * (1.0 / N)
        mp = jnp.sum(nf_p, axis=0, keepdims=True) * (1.0 / N)
        gfp_o[...] = (jnp.dot(md, wgd_ref[...], preferred_element_type=jnp.float32)
                      + jnp.dot(mp, wgp_ref[...], preferred_element_type=jnp.float32))
    return pl.pallas_call(
        body,
        out_shape=(
            jax.ShapeDtypeStruct((N, 16), jnp.float32),
            jax.ShapeDtypeStruct((N, 16), jnp.float32),
            jax.ShapeDtypeStruct((N, 128), jnp.float32),
            jax.ShapeDtypeStruct((1, 128), jnp.float32),
        ),
    )(aggdp, xwn, wnd2, wnp2, wgd, wgp, wss, wns1)


def _tc_post(aggs, xwns, wns2, wgs, gfp):
    def body(aggs_ref, xwns_ref, wns2_ref, wgs_ref, gfp_ref, nf_o, gf_o):
        a = aggs_ref[...]
        agg = a[0, :N] + a[1, :N]
        nf_s = _relu(xwns_ref[...] + jnp.dot(
            agg, wns2_ref[...], preferred_element_type=jnp.float32))
        nf_o[...] = nf_s
        ms = jnp.sum(nf_s, axis=0, keepdims=True) * (1.0 / N)
        gf_o[...] = gfp_ref[...] + jnp.dot(
            ms, wgs_ref[...], preferred_element_type=jnp.float32)
    return pl.pallas_call(
        body,
        out_shape=(
            jax.ShapeDtypeStruct((N, 128), jnp.float32),
            jax.ShapeDtypeStruct((1, 128), jnp.float32),
        ),
    )(aggs, xwns, wns2, wgs, gfp)


def kernel(x, edge_index, edge_attr,
           We_d, Wn_d, Wg_d, We_p, Wn_p, Wg_p, We_s, Wn_s, Wg_s):
    ei = edge_index.astype(jnp.int32)
    idx_real = jnp.concatenate(
        [ei[0].reshape(E // CHUNK, NSUB, GSUB),
         ei[1].reshape(E // CHUNK, NSUB, GSUB)], axis=1)
    npadchunks = NCHP - E // CHUNK
    idx_fill = jnp.concatenate(
        [jnp.zeros((npadchunks, NSUB, GSUB), jnp.int32),
         jnp.full((npadchunks, NSUB, GSUB), NPAD - 1, jnp.int32)], axis=1)
    idx_r = jnp.concatenate([idx_real, idx_fill], axis=0)
    ea_r = edge_attr.reshape(ER, 128)

    wa = jnp.concatenate([Wn_d[:128], Wn_p[:128]], axis=1)
    wb = jnp.concatenate([We_d[:128], We_p[:128]], axis=1)
    wc = jnp.concatenate([We_d[128:256], We_p[128:256]], axis=1)
    eye8 = jnp.eye(8, dtype=jnp.float32)
    w3d = jnp.kron(eye8, We_d[256:])
    w3p = jnp.kron(eye8, We_p[256:])
    w3s = jnp.kron(eye8, We_s[256:])

    xwn, xs_tab, xd_tab = _tc_node_pre(x, wa, wb, wc)
    ea_d, ea_p = _tc_edge_bd2(ea_r, w3d, w3p)

    z32 = jnp.zeros((NPAD, 32), jnp.float32)
    ef_in, agg_dp = _edge_stage2(idx_r, xs_tab, xd_tab, ea_d, ea_p, z32)

    wss = jnp.concatenate([We_s[:128], We_s[128:256]], axis=1)
    ns_tab, nd_tab, xwns, gfp = _tc_mid(
        agg_dp, xwn, Wn_d[128:], Wn_p[128:], Wg_d, Wg_p, wss, Wn_s[:128])
    es_in = _tc_edge_bd1(ef_in, w3s)

    z16 = jnp.zeros((NPAD, 16), jnp.float32)
    ef_s, agg_s = _edge_stage1(idx_r, ns_tab, nd_tab, es_in, z16)

    nf_s, gf = _tc_post(agg_s, xwns, Wn_s[128:], Wg_s, gfp)
    return nf_s, ef_s[:ER].reshape(E, DE), gf.reshape(D)


# speedup vs baseline: 9.2738x; 1.0689x over previous
import functools

import jax
import jax.numpy as jnp
from jax import lax
from jax.experimental import pallas as pl
from jax.experimental.pallas import tpu as pltpu
from jax.experimental.pallas import tpu_sc as plsc

N = 10000
E = 320000
D = 128
DE = 16
NC = 2
NS = 16
NW = NC * NS
GSUB = 128
CHUNK = 256
NSUB = CHUNK // GSUB
RPC = CHUNK // 8
ROWS_PER_SUB = 632
NPAD = ROWS_PER_SUB * NS
ER = E // 8
KPW = 40
EPAD = KPW * NW * CHUNK
NCHP = EPAD // CHUNK
ERP = EPAD // 8


def _relu(v):
    return jnp.maximum(v, 0.0)


def _make_edge_stage(nb):
    W = 16 * nb
    mesh = plsc.VectorSubcoreMesh(
        core_axis_name="c", subcore_axis_name="s", num_cores=NC, num_subcores=NS
    )

    ea_scratch = [pltpu.VMEM((2, RPC, 128), jnp.float32) for _ in range(nb)]
    if nb == 2:
        ef_out_type = jax.ShapeDtypeStruct((ERP, 128), jnp.float32)
        efo_scratch = pltpu.VMEM((2, RPC, 128), jnp.float32)
    else:
        ef_out_type = jax.ShapeDtypeStruct((16, E), jnp.float32)
        efo_scratch = pltpu.VMEM((2, 16, CHUNK), jnp.float32)

    @functools.partial(
        pl.kernel,
        mesh=mesh,
        compiler_params=pltpu.CompilerParams(use_tc_tiling_on_sc=False,
                                             needs_layout_passes=False),
        out_type=(
            ef_out_type,
            jax.ShapeDtypeStruct((NC, NPAD, W), jnp.float32),
        ),
        scratch_types=[
            pltpu.VMEM((KPW, 2 * NSUB, GSUB), jnp.int32),
            pltpu.VMEM((2, CHUNK, W), jnp.float32),
            pltpu.VMEM((2, CHUNK, W), jnp.float32),
            *ea_scratch,
            pltpu.VMEM((2, CHUNK, W), jnp.float32),
            efo_scratch,
            pltpu.VMEM_SHARED((NPAD, W), jnp.float32),
            pltpu.SemaphoreType.DMA,
            pltpu.SemaphoreType.DMA,
            pltpu.SemaphoreType.DMA,
            pltpu.SemaphoreType.DMA,
        ],
    )
    def stage(idx_r, stab, dtab, *rest):
        if nb == 2:
            (ea0, ea1, zrows, ef_out, agg_out, idx_all, sv, dv, eav0, eav1,
             efv, efo, agg_sh, sg0, sg1, so0, so1) = rest
            eas, eavs = (ea0, ea1), (eav0, eav1)
        else:
            (ea0, zrows, ef_out, agg_out, idx_all, sv, dv, eav0,
             efv, efo, agg_sh, sg0, sg1, so0, so1) = rest
            eas, eavs = (ea0,), (eav0,)
        semg = (sg0, sg1)
        semo = (so0, so1)
        cid = lax.axis_index("c")
        sid = lax.axis_index("s")
        wid = sid * NC + cid
        q0 = wid * KPW
        r0 = sid * ROWS_PER_SUB
        pltpu.sync_copy(zrows.at[pl.ds(r0, ROWS_PER_SUB)],
                        agg_sh.at[pl.ds(r0, ROWS_PER_SUB)])
        pltpu.sync_copy(idx_r.at[pl.ds(q0, KPW)], idx_all)
        plsc.subcore_barrier()

        def fire_in(i, b):
            for j in range(NSUB):
                pltpu.async_copy(stab.at[idx_all.at[i, j]],
                                 sv.at[b, pl.ds(j * GSUB, GSUB)], semg[b])
                pltpu.async_copy(dtab.at[idx_all.at[i, NSUB + j]],
                                 dv.at[b, pl.ds(j * GSUB, GSUB)], semg[b])
            for t in range(nb):
                pltpu.async_copy(eas[t].at[pl.ds((q0 + i) * RPC, RPC)],
                                 eavs[t].at[b], semg[b])

        def drain_in(i, b):
            for j in range(NSUB):
                pltpu.make_async_copy(stab.at[idx_all.at[i, j]],
                                      sv.at[b, pl.ds(j * GSUB, GSUB)],
                                      semg[b]).wait()
                pltpu.make_async_copy(dtab.at[idx_all.at[i, NSUB + j]],
                                      dv.at[b, pl.ds(j * GSUB, GSUB)],
                                      semg[b]).wait()
            for t in range(nb):
                pltpu.make_async_copy(eas[t].at[pl.ds((q0 + i) * RPC, RPC)],
                                      eavs[t].at[b], semg[b]).wait()

        def fire_out(i, b):
            for j in range(NSUB):
                pltpu.sync_copy(efv.at[b, pl.ds(j * GSUB, GSUB)],
                                agg_sh.at[idx_all.at[i, NSUB + j]], add=True)
            if nb == 2:
                pltpu.async_copy(efo.at[b],
                                 ef_out.at[pl.ds((q0 + i) * RPC, RPC)],
                                 semo[b])
            else:
                @pl.when(q0 + i < E // CHUNK)
                def _():
                    pltpu.async_copy(
                        efo.at[b],
                        ef_out.at[:, pl.ds((q0 + i) * CHUNK, CHUNK)], semo[b])

        def drain_out(i, b):
            if nb == 2:
                pltpu.make_async_copy(efo.at[b],
                                      ef_out.at[pl.ds((q0 + i) * RPC, RPC)],
                                      semo[b]).wait()
            else:
                @pl.when(q0 + i < E // CHUNK)
                def _():
                    pltpu.make_async_copy(
                        efo.at[b],
                        ef_out.at[:, pl.ds((q0 + i) * CHUNK, CHUNK)],
                        semo[b]).wait()

        lanes = lax.iota(jnp.int32, 16)

        def compute(i, b):
            def row_body(rr, c2):
                for jj in range(8):
                    e = rr * 8 + jj
                    acc = None
                    for t in range(nb):
                        s = sv[b, e, pl.ds(16 * t, 16)]
                        d = dv[b, e, pl.ds(16 * t, 16)]
                        a = eavs[t][b, rr, pl.ds(16 * jj, 16)]
                        eft = _relu(s + d + a)
                        efv[b, e, pl.ds(16 * t, 16)] = eft
                        acc = eft if acc is None else acc + eft
                    if nb == 2:
                        efo[b, rr, pl.ds(16 * jj, 16)] = acc
                    else:
                        plsc.store_scatter(
                            efo.at[b], [lanes, jnp.full((16,), e, jnp.int32)],
                            acc)
                return c2
            lax.fori_loop(0, RPC, row_body, 0)

        fire_in(0, 0)
        fire_in(1, 1)
        drain_in(0, 0)
        compute(0, 0)
        fire_out(0, 0)
        fire_in(2, 0)
        drain_in(1, 1)
        compute(1, 1)
        fire_out(1, 1)
        fire_in(3, 1)

        def pair_body(kk, carry):
            for b in range(2):
                i = 2 * kk + b
                drain_in(i, b)
                drain_out(i - 2, b)
                compute(i, b)
                fire_out(i, b)
                fire_in(i + 2, b)
            return carry
        lax.fori_loop(1, KPW // 2 - 1, pair_body, 0)

        for b in range(2):
            i = KPW - 2 + b
            drain_in(i, b)
            drain_out(i - 2, b)
            compute(i, b)
            fire_out(i, b)
        for b in range(2):
            drain_out(KPW - 2 + b, b)

        plsc.subcore_barrier()
        pltpu.sync_copy(agg_sh.at[pl.ds(r0, ROWS_PER_SUB)],
                        agg_out.at[cid, pl.ds(r0, ROWS_PER_SUB)])

    return stage


_edge_stage2 = _make_edge_stage(2)
_edge_stage1 = _make_edge_stage(1)


def _tc_node_pre(x, wa, wb, wc):
    def body(x_ref, wa_ref, wb_ref, wc_ref, o1, o2, o3):
        xv = x_ref[...]
        o1[...] = jnp.dot(xv, wa_ref[...], preferred_element_type=jnp.float32)
        o2[...] = jnp.dot(xv, wb_ref[...], preferred_element_type=jnp.float32)
        o3[...] = jnp.dot(xv, wc_ref[...], preferred_element_type=jnp.float32)
    return pl.pallas_call(
        body,
        out_shape=(
            jax.ShapeDtypeStruct((N, 256), jnp.float32),
            jax.ShapeDtypeStruct((N, 32), jnp.float32),
            jax.ShapeDtypeStruct((N, 32), jnp.float32),
        ),
    )(x, wa, wb, wc)


def _tc_edge_bd2(a, w0, w1):
    BR = 8000
    def body(a_ref, w0_ref, w1_ref, o0_ref, o1_ref):
        av = a_ref[...]
        o0_ref[...] = jnp.dot(av, w0_ref[...],
                              preferred_element_type=jnp.float32)
        o1_ref[...] = jnp.dot(av, w1_ref[...],
                              preferred_element_type=jnp.float32)
    return pl.pallas_call(
        body,
        grid=(ER // BR,),
        in_specs=[pl.BlockSpec((BR, 128), lambda i: (i, 0)),
                  pl.BlockSpec((128, 128), lambda i: (0, 0)),
                  pl.BlockSpec((128, 128), lambda i: (0, 0))],
        out_specs=(pl.BlockSpec((BR, 128), lambda i: (i, 0)),
                   pl.BlockSpec((BR, 128), lambda i: (i, 0))),
        out_shape=(jax.ShapeDtypeStruct((ERP, 128), jnp.float32),
                   jax.ShapeDtypeStruct((ERP, 128), jnp.float32)),
    )(a, w0, w1)


def _tc_edge_bd1(a, w0):
    BR = 8000
    def body(a_ref, w0_ref, o0_ref):
        o0_ref[...] = jnp.dot(a_ref[...], w0_ref[...],
                              preferred_element_type=jnp.float32)
    return pl.pallas_call(
        body,
        grid=(ER // BR,),
        in_specs=[pl.BlockSpec((BR, 128), lambda i: (i, 0)),
                  pl.BlockSpec((128, 128), lambda i: (0, 0))],
        out_specs=pl.BlockSpec((BR, 128), lambda i: (i, 0)),
        out_shape=jax.ShapeDtypeStruct((ERP, 128), jnp.float32),
    )(a, w0)


def _tc_mid(aggdp, xwn, wnd2, wnp2, wgd, wgp, wss, wns1):
    def body(agg_ref, xwn_ref, wnd2_ref, wnp2_ref, wgd_ref, wgp_ref,
             wss_ref, wns1_ref, ns_o, nd_o, xwns_o, gfp_o):
        a = agg_ref[...]
        agg = a[0, :N] + a[1, :N]
        xw = xwn_ref[...]
        nf_d = _relu(xw[:, 0:128] + jnp.dot(
            agg[:, 0:16], wnd2_ref[...], preferred_element_type=jnp.float32))
        nf_p = _relu(xw[:, 128:256] + jnp.dot(
            agg[:, 16:32], wnp2_ref[...], preferred_element_type=jnp.float32))
        nf = nf_d + nf_p
        nsnd = jnp.dot(nf, wss_ref[...], preferred_element_type=jnp.float32)
        ns_o[...] = nsnd[:, 0:16]
        nd_o[...] = nsnd[:, 16:32]
        xwns_o[...] = jnp.dot(nf, wns1_ref[...],
                              preferred_element_type=jnp.float32)
        md = jnp.sum(nf_d, axis=0, keepdims=True) * (1.0 / N)
        mp = jnp.sum(nf_p, axis=0, keepdims=True) * (1.0 / N)
        gfp_o[...] = (jnp.dot(md, wgd_ref[...], preferred_element_type=jnp.float32)
                      + jnp.dot(mp, wgp_ref[...], preferred_element_type=jnp.float32))
    return pl.pallas_call(
        body,
        out_shape=(
            jax.ShapeDtypeStruct((N, 16), jnp.float32),
            jax.ShapeDtypeStruct((N, 16), jnp.float32),
            jax.ShapeDtypeStruct((N, 128), jnp.float32),
            jax.ShapeDtypeStruct((1, 128), jnp.float32),
        ),
    )(aggdp, xwn, wnd2, wnp2, wgd, wgp, wss, wns1)


def _tc_post(aggs, xwns, wns2, wgs, gfp):
    def body(aggs_ref, xwns_ref, wns2_ref, wgs_ref, gfp_ref, nf_o, gf_o):
        a = aggs_ref[...]
        agg = a[0, :N] + a[1, :N]
        nf_s = _relu(xwns_ref[...] + jnp.dot(
            agg, wns2_ref[...], preferred_element_type=jnp.float32))
        nf_o[...] = nf_s
        ms = jnp.sum(nf_s, axis=0, keepdims=True) * (1.0 / N)
        gf_o[...] = gfp_ref[...] + jnp.dot(
            ms, wgs_ref[...], preferred_element_type=jnp.float32)
    return pl.pallas_call(
        body,
        out_shape=(
            jax.ShapeDtypeStruct((N, 128), jnp.float32),
            jax.ShapeDtypeStruct((1, 128), jnp.float32),
        ),
    )(aggs, xwns, wns2, wgs, gfp)


def kernel(x, edge_index, edge_attr,
           We_d, Wn_d, Wg_d, We_p, Wn_p, Wg_p, We_s, Wn_s, Wg_s):
    ei = edge_index.astype(jnp.int32)
    idx_real = jnp.concatenate(
        [ei[0].reshape(E // CHUNK, NSUB, GSUB),
         ei[1].reshape(E // CHUNK, NSUB, GSUB)], axis=1)
    npadchunks = NCHP - E // CHUNK
    idx_fill = jnp.concatenate(
        [jnp.zeros((npadchunks, NSUB, GSUB), jnp.int32),
         jnp.full((npadchunks, NSUB, GSUB), NPAD - 1, jnp.int32)], axis=1)
    idx_r = jnp.concatenate([idx_real, idx_fill], axis=0)
    ea_r = edge_attr.reshape(ER, 128)

    wa = jnp.concatenate([Wn_d[:128], Wn_p[:128]], axis=1)
    wb = jnp.concatenate([We_d[:128], We_p[:128]], axis=1)
    wc = jnp.concatenate([We_d[128:256], We_p[128:256]], axis=1)
    eye8 = jnp.eye(8, dtype=jnp.float32)
    w3d = jnp.kron(eye8, We_d[256:])
    w3p = jnp.kron(eye8, We_p[256:])
    w3s = jnp.kron(eye8, We_s[256:])

    xwn, xs_tab, xd_tab = _tc_node_pre(x, wa, wb, wc)
    ea_d, ea_p = _tc_edge_bd2(ea_r, w3d, w3p)

    z32 = jnp.zeros((NPAD, 32), jnp.float32)
    ef_in, agg_dp = _edge_stage2(idx_r, xs_tab, xd_tab, ea_d, ea_p, z32)

    wss = jnp.concatenate([We_s[:128], We_s[128:256]], axis=1)
    ns_tab, nd_tab, xwns, gfp = _tc_mid(
        agg_dp, xwn, Wn_d[128:], Wn_p[128:], Wg_d, Wg_p, wss, Wn_s[:128])
    es_in = _tc_edge_bd1(ef_in, w3s)

    z16 = jnp.zeros((NPAD, 16), jnp.float32)
    ef_s, agg_s = _edge_stage1(idx_r, ns_tab, nd_tab, es_in, z16)

    nf_s, gf = _tc_post(agg_s, xwns, Wn_s[128:], Wg_s, gfp)
    return nf_s, ef_s.T, gf.reshape(D)


# speedup vs baseline: 9.5300x; 1.0276x over previous
import functools

import jax
import jax.numpy as jnp
from jax import lax
from jax.experimental import pallas as pl
from jax.experimental.pallas import tpu as pltpu
from jax.experimental.pallas import tpu_sc as plsc

N = 10000
E = 320000
D = 128
DE = 16
NC = 2
NS = 16
NW = NC * NS
GSUB = 128
CHUNK = 256
NSUB = CHUNK // GSUB
RPC = CHUNK // 8
ROWS_PER_SUB = 632
NPAD = ROWS_PER_SUB * NS
ER = E // 8
KPW = 40
EPAD = KPW * NW * CHUNK
NCHP = EPAD // CHUNK
ERP = EPAD // 8


def _relu(v):
    return jnp.maximum(v, 0.0)


def _make_edge_stage(nb):
    W = 16 * nb
    mesh = plsc.VectorSubcoreMesh(
        core_axis_name="c", subcore_axis_name="s", num_cores=NC, num_subcores=NS
    )

    ea_scratch = [pltpu.VMEM((2, RPC, 128), jnp.float32) for _ in range(nb)]
    if nb == 2:
        ef_out_type = jax.ShapeDtypeStruct((ERP, 128), jnp.float32)
        efo_scratch = pltpu.VMEM((2, RPC, 128), jnp.float32)
    else:
        ef_out_type = jax.ShapeDtypeStruct((16, E), jnp.float32)
        efo_scratch = pltpu.VMEM((2, 16, CHUNK), jnp.float32)

    @functools.partial(
        pl.kernel,
        mesh=mesh,
        compiler_params=pltpu.CompilerParams(use_tc_tiling_on_sc=False,
                                             needs_layout_passes=False),
        out_type=(
            ef_out_type,
            jax.ShapeDtypeStruct((NC, NPAD, W), jnp.float32),
        ),
        scratch_types=[
            pltpu.VMEM((KPW, 2 * NSUB, GSUB), jnp.int32),
            pltpu.VMEM((2, CHUNK, W), jnp.float32),
            pltpu.VMEM((2, CHUNK, W), jnp.float32),
            *ea_scratch,
            pltpu.VMEM((2, CHUNK, W), jnp.float32),
            efo_scratch,
            pltpu.VMEM_SHARED((NPAD, W), jnp.float32),
            pltpu.SemaphoreType.DMA,
            pltpu.SemaphoreType.DMA,
            pltpu.SemaphoreType.DMA,
            pltpu.SemaphoreType.DMA,
            pltpu.SemaphoreType.DMA,
            pltpu.SemaphoreType.DMA,
        ],
    )
    def stage(idx_r, stab, dtab, *rest):
        if nb == 2:
            (ea0, ea1, zrows, ef_out, agg_out, idx_all, sv, dv, eav0, eav1,
             efv, efo, agg_sh, sg0, sg1, so0, so1, sa0, sa1) = rest
            eas, eavs = (ea0, ea1), (eav0, eav1)
        else:
            (ea0, zrows, ef_out, agg_out, idx_all, sv, dv, eav0,
             efv, efo, agg_sh, sg0, sg1, so0, so1, sa0, sa1) = rest
            eas, eavs = (ea0,), (eav0,)
        semg = (sg0, sg1)
        semo = (so0, so1)
        sema = (sa0, sa1)
        cid = lax.axis_index("c")
        sid = lax.axis_index("s")
        wid = sid * NC + cid
        q0 = wid * KPW
        r0 = sid * ROWS_PER_SUB
        pltpu.sync_copy(zrows.at[pl.ds(r0, ROWS_PER_SUB)],
                        agg_sh.at[pl.ds(r0, ROWS_PER_SUB)])
        pltpu.sync_copy(idx_r.at[pl.ds(q0, KPW)], idx_all)
        plsc.subcore_barrier()

        def fire_in(i, b):
            for j in range(NSUB):
                pltpu.async_copy(stab.at[idx_all.at[i, j]],
                                 sv.at[b, pl.ds(j * GSUB, GSUB)], semg[b])
                pltpu.async_copy(dtab.at[idx_all.at[i, NSUB + j]],
                                 dv.at[b, pl.ds(j * GSUB, GSUB)], semg[b])
            for t in range(nb):
                pltpu.async_copy(eas[t].at[pl.ds((q0 + i) * RPC, RPC)],
                                 eavs[t].at[b], semg[b])

        def drain_in(i, b):
            for j in range(NSUB):
                pltpu.make_async_copy(stab.at[idx_all.at[i, j]],
                                      sv.at[b, pl.ds(j * GSUB, GSUB)],
                                      semg[b]).wait()
                pltpu.make_async_copy(dtab.at[idx_all.at[i, NSUB + j]],
                                      dv.at[b, pl.ds(j * GSUB, GSUB)],
                                      semg[b]).wait()
            for t in range(nb):
                pltpu.make_async_copy(eas[t].at[pl.ds((q0 + i) * RPC, RPC)],
                                      eavs[t].at[b], semg[b]).wait()

        def drain_adds(i, b):
            for j in range(NSUB):
                pltpu.make_async_copy(efv.at[b, pl.ds(j * GSUB, GSUB)],
                                      agg_sh.at[idx_all.at[i, NSUB + j]],
                                      sema[b]).wait()

        def fire_out(i, b, drain_prev=True):
            if drain_prev:
                drain_adds(i - 1, 1 - b)
            for j in range(NSUB):
                pltpu.async_copy(efv.at[b, pl.ds(j * GSUB, GSUB)],
                                 agg_sh.at[idx_all.at[i, NSUB + j]], sema[b],
                                 add=True)
            if nb == 2:
                pltpu.async_copy(efo.at[b],
                                 ef_out.at[pl.ds((q0 + i) * RPC, RPC)],
                                 semo[b])
            else:
                @pl.when(q0 + i < E // CHUNK)
                def _():
                    pltpu.async_copy(
                        efo.at[b],
                        ef_out.at[:, pl.ds((q0 + i) * CHUNK, CHUNK)], semo[b])

        def drain_out(i, b):
            if nb == 2:
                pltpu.make_async_copy(efo.at[b],
                                      ef_out.at[pl.ds((q0 + i) * RPC, RPC)],
                                      semo[b]).wait()
            else:
                @pl.when(q0 + i < E // CHUNK)
                def _():
                    pltpu.make_async_copy(
                        efo.at[b],
                        ef_out.at[:, pl.ds((q0 + i) * CHUNK, CHUNK)],
                        semo[b]).wait()

        lanes = lax.iota(jnp.int32, 16)

        def compute(i, b):
            def row_body(rr, c2):
                for jj in range(8):
                    e = rr * 8 + jj
                    acc = None
                    for t in range(nb):
                        s = sv[b, e, pl.ds(16 * t, 16)]
                        d = dv[b, e, pl.ds(16 * t, 16)]
                        a = eavs[t][b, rr, pl.ds(16 * jj, 16)]
                        eft = _relu(s + d + a)
                        efv[b, e, pl.ds(16 * t, 16)] = eft
                        acc = eft if acc is None else acc + eft
                    if nb == 2:
                        efo[b, rr, pl.ds(16 * jj, 16)] = acc
                    else:
                        plsc.store_scatter(
                            efo.at[b], [lanes, jnp.full((16,), e, jnp.int32)],
                            acc)
                return c2
            lax.fori_loop(0, RPC, row_body, 0)

        fire_in(0, 0)
        fire_in(1, 1)
        drain_in(0, 0)
        compute(0, 0)
        fire_out(0, 0, drain_prev=False)
        fire_in(2, 0)
        drain_in(1, 1)
        compute(1, 1)
        fire_out(1, 1)
        fire_in(3, 1)

        def pair_body(kk, carry):
            for b in range(2):
                i = 2 * kk + b
                drain_in(i, b)
                drain_out(i - 2, b)
                compute(i, b)
                fire_out(i, b)
                fire_in(i + 2, b)
            return carry
        lax.fori_loop(1, KPW // 2 - 1, pair_body, 0)

        for b in range(2):
            i = KPW - 2 + b
            drain_in(i, b)
            drain_out(i - 2, b)
            compute(i, b)
            fire_out(i, b)
        for b in range(2):
            drain_out(KPW - 2 + b, b)
        drain_adds(KPW - 1, 1)

        plsc.subcore_barrier()
        pltpu.sync_copy(agg_sh.at[pl.ds(r0, ROWS_PER_SUB)],
                        agg_out.at[cid, pl.ds(r0, ROWS_PER_SUB)])

    return stage


_edge_stage2 = _make_edge_stage(2)
_edge_stage1 = _make_edge_stage(1)


def _tc_node_pre(x, wa, wb, wc):
    def body(x_ref, wa_ref, wb_ref, wc_ref, o1, o2, o3):
        xv = x_ref[...]
        o1[...] = jnp.dot(xv, wa_ref[...], preferred_element_type=jnp.float32)
        o2[...] = jnp.dot(xv, wb_ref[...], preferred_element_type=jnp.float32)
        o3[...] = jnp.dot(xv, wc_ref[...], preferred_element_type=jnp.float32)
    return pl.pallas_call(
        body,
        out_shape=(
            jax.ShapeDtypeStruct((N, 256), jnp.float32),
            jax.ShapeDtypeStruct((N, 32), jnp.float32),
            jax.ShapeDtypeStruct((N, 32), jnp.float32),
        ),
    )(x, wa, wb, wc)


def _tc_edge_bd2(a, w0, w1):
    BR = 8000
    def body(a_ref, w0_ref, w1_ref, o0_ref, o1_ref):
        av = a_ref[...]
        o0_ref[...] = jnp.dot(av, w0_ref[...],
                              preferred_element_type=jnp.float32)
        o1_ref[...] = jnp.dot(av, w1_ref[...],
                              preferred_element_type=jnp.float32)
    return pl.pallas_call(
        body,
        grid=(ER // BR,),
        in_specs=[pl.BlockSpec((BR, 128), lambda i: (i, 0)),
                  pl.BlockSpec((128, 128), lambda i: (0, 0)),
                  pl.BlockSpec((128, 128), lambda i: (0, 0))],
        out_specs=(pl.BlockSpec((BR, 128), lambda i: (i, 0)),
                   pl.BlockSpec((BR, 128), lambda i: (i, 0))),
        out_shape=(jax.ShapeDtypeStruct((ERP, 128), jnp.float32),
                   jax.ShapeDtypeStruct((ERP, 128), jnp.float32)),
    )(a, w0, w1)


def _tc_edge_bd1(a, w0):
    BR = 8000
    def body(a_ref, w0_ref, o0_ref):
        o0_ref[...] = jnp.dot(a_ref[...], w0_ref[...],
                              preferred_element_type=jnp.float32)
    return pl.pallas_call(
        body,
        grid=(ER // BR,),
        in_specs=[pl.BlockSpec((BR, 128), lambda i: (i, 0)),
                  pl.BlockSpec((128, 128), lambda i: (0, 0))],
        out_specs=pl.BlockSpec((BR, 128), lambda i: (i, 0)),
        out_shape=jax.ShapeDtypeStruct((ERP, 128), jnp.float32),
    )(a, w0)


def _tc_mid(aggdp, xwn, wnd2, wnp2, wgd, wgp, wss, wns1):
    def body(agg_ref, xwn_ref, wnd2_ref, wnp2_ref, wgd_ref, wgp_ref,
             wss_ref, wns1_ref, ns_o, nd_o, xwns_o, gfp_o):
        a = agg_ref[...]
        agg = a[0, :N] + a[1, :N]
        xw = xwn_ref[...]
        nf_d = _relu(xw[:, 0:128] + jnp.dot(
            agg[:, 0:16], wnd2_ref[...], preferred_element_type=jnp.float32))
        nf_p = _relu(xw[:, 128:256] + jnp.dot(
            agg[:, 16:32], wnp2_ref[...], preferred_element_type=jnp.float32))
        nf = nf_d + nf_p
        nsnd = jnp.dot(nf, wss_ref[...], preferred_element_type=jnp.float32)
        ns_o[...] = nsnd[:, 0:16]
        nd_o[...] = nsnd[:, 16:32]
        xwns_o[...] = jnp.dot(nf, wns1_ref[...],
                              preferred_element_type=jnp.float32)
        md = jnp.sum(nf_d, axis=0, keepdims=True) * (1.0 / N)
        mp = jnp.sum(nf_p, axis=0, keepdims=True) * (1.0 / N)
        gfp_o[...] = (jnp.dot(md, wgd_ref[...], preferred_element_type=jnp.float32)
                      + jnp.dot(mp, wgp_ref[...], preferred_element_type=jnp.float32))
    return pl.pallas_call(
        body,
        out_shape=(
            jax.ShapeDtypeStruct((N, 16), jnp.float32),
            jax.ShapeDtypeStruct((N, 16), jnp.float32),
            jax.ShapeDtypeStruct((N, 128), jnp.float32),
            jax.ShapeDtypeStruct((1, 128), jnp.float32),
        ),
    )(aggdp, xwn, wnd2, wnp2, wgd, wgp, wss, wns1)


def _tc_post(aggs, xwns, wns2, wgs, gfp):
    def body(aggs_ref, xwns_ref, wns2_ref, wgs_ref, gfp_ref, nf_o, gf_o):
        a = aggs_ref[...]
        agg = a[0, :N] + a[1, :N]
        nf_s = _relu(xwns_ref[...] + jnp.dot(
            agg, wns2_ref[...], preferred_element_type=jnp.float32))
        nf_o[...] = nf_s
        ms = jnp.sum(nf_s, axis=0, keepdims=True) * (1.0 / N)
        gf_o[...] = gfp_ref[...] + jnp.dot(
            ms, wgs_ref[...], preferred_element_type=jnp.float32)
    return pl.pallas_call(
        body,
        out_shape=(
            jax.ShapeDtypeStruct((N, 128), jnp.float32),
            jax.ShapeDtypeStruct((1, 128), jnp.float32),
        ),
    )(aggs, xwns, wns2, wgs, gfp)


def kernel(x, edge_index, edge_attr,
           We_d, Wn_d, Wg_d, We_p, Wn_p, Wg_p, We_s, Wn_s, Wg_s):
    ei = edge_index.astype(jnp.int32)
    idx_real = jnp.concatenate(
        [ei[0].reshape(E // CHUNK, NSUB, GSUB),
         ei[1].reshape(E // CHUNK, NSUB, GSUB)], axis=1)
    npadchunks = NCHP - E // CHUNK
    idx_fill = jnp.concatenate(
        [jnp.zeros((npadchunks, NSUB, GSUB), jnp.int32),
         jnp.full((npadchunks, NSUB, GSUB), NPAD - 1, jnp.int32)], axis=1)
    idx_r = jnp.concatenate([idx_real, idx_fill], axis=0)
    ea_r = edge_attr.reshape(ER, 128)

    wa = jnp.concatenate([Wn_d[:128], Wn_p[:128]], axis=1)
    wb = jnp.concatenate([We_d[:128], We_p[:128]], axis=1)
    wc = jnp.concatenate([We_d[128:256], We_p[128:256]], axis=1)
    eye8 = jnp.eye(8, dtype=jnp.float32)
    w3d = jnp.kron(eye8, We_d[256:])
    w3p = jnp.kron(eye8, We_p[256:])
    w3s = jnp.kron(eye8, We_s[256:])

    xwn, xs_tab, xd_tab = _tc_node_pre(x, wa, wb, wc)
    ea_d, ea_p = _tc_edge_bd2(ea_r, w3d, w3p)

    z32 = jnp.zeros((NPAD, 32), jnp.float32)
    ef_in, agg_dp = _edge_stage2(idx_r, xs_tab, xd_tab, ea_d, ea_p, z32)

    wss = jnp.concatenate([We_s[:128], We_s[128:256]], axis=1)
    ns_tab, nd_tab, xwns, gfp = _tc_mid(
        agg_dp, xwn, Wn_d[128:], Wn_p[128:], Wg_d, Wg_p, wss, Wn_s[:128])
    es_in = _tc_edge_bd1(ef_in, w3s)

    z16 = jnp.zeros((NPAD, 16), jnp.float32)
    ef_s, agg_s = _edge_stage1(idx_r, ns_tab, nd_tab, es_in, z16)

    nf_s, gf = _tc_post(agg_s, xwns, Wn_s[128:], Wg_s, gfp)
    return nf_s, ef_s.T, gf.reshape(D)
